# baseline (speedup 1.0000x reference)
import functools

import jax
import jax.numpy as jnp
import numpy as np
from jax import lax
from jax.experimental import pallas as pl
from jax.experimental.pallas import tpu as pltpu
from jax.experimental.pallas import tpu_sc as plsc

D = 128
R = 16
T_NODES = 32
CH = 64
NC = 2
NS = 16
NW = NC * NS
ACC_W = 9 * D
NBLK = 256
GBLK = 1000


def _swish(v):
    return v * jax.nn.sigmoid(v)


def _bf16_bits(y):
    u = lax.bitcast_convert_type(y, jnp.int32)
    return (u + 0x7FFF + ((u >> 16) & 1)) >> 16


def _pack_pairs(a, b, c):
    w1 = (_bf16_bits(a) & 0xFFFF) | (_bf16_bits(b) << 16)
    w2 = _bf16_bits(c) & 0xFFFF
    return jnp.concatenate([w1, w2], axis=1)


def _resmlp(v, w1t, b1, w2t, b2, wft, bf):
    y = _swish(v)
    y = jnp.dot(y, w1t, preferred_element_type=jnp.float32) + b1
    y = _swish(y)
    y = jnp.dot(y, w2t, preferred_element_type=jnp.float32) + b2
    v = v + y
    return jnp.dot(_swish(v), wft, preferred_element_type=jnp.float32) + bf


def _nodes_body(x_ref, *refs):
    (xw1, xb1, xw2, xb2, xwf, xbf,
     sw1, sb1, sw2, sb2, swf, sbf,
     pw1, pb1, pw2, pb2, pwf, pbf,
     dw1, db1, dw2, db2, dwf, dbf,
     xx_ref, xcat_ref) = refs
    xb = x_ref[...]
    xx = _resmlp(xb, xw1[...], xb1[...], xw2[...], xb2[...], xwf[...], xbf[...])
    xs = _resmlp(xb, sw1[...], sb1[...], sw2[...], sb2[...], swf[...], sbf[...])
    xp = _resmlp(xb, pw1[...], pb1[...], pw2[...], pb2[...], pwf[...], pbf[...])
    xd = _resmlp(xb, dw1[...], db1[...], dw2[...], db2[...], dwf[...], dbf[...])
    xx_ref[...] = xx
    xcat_ref[...] = _pack_pairs(xs, xp, xd)


def _gcat_body(rbf_ref, wg_ref, gcat_ref):
    g = jnp.dot(rbf_ref[...], wg_ref[...],
                preferred_element_type=jnp.float32)
    gcat_ref[...] = _pack_pairs(g[:, :D], g[:, D:2 * D], g[:, 2 * D:])


def _sc_body(n_pad, p_pad, nt,
             offs_hbm, idxj_hbm, xcat_hbm, gcat_hbm, pd_hbm, out_hbm,
             offs_v, ij0, ij1, xg0, xg1, gg0, gg1, pd0, pd1, acc_v,
             dsem0, dsem1, jsem0, jsem1):
    del n_pad
    np_edges = p_pad
    w = lax.axis_index("s") * NC + lax.axis_index("c")
    per_w = nt // NW
    pltpu.sync_copy(offs_hbm, offs_v)
    zv = jnp.zeros((16,), jnp.float32)
    ij = (ij0, ij1)
    xg = (xg0, xg1)
    gg = (gg0, gg1)
    pdb = (pd0, pd1)
    dsem = (dsem0, dsem1)
    jsem = (jsem0, jsem1)

    def subtile(t, carry):
        st = t * NW + w

        def zbody(z, c2):
            acc_v[pl.ds(z * 16, 16)] = zv
            return c2
        lax.fori_loop(0, (T_NODES * ACC_W) // 16, zbody, 0)

        e_lo = offs_v[pl.ds(st, 16)][0]
        e_hi = offs_v[pl.ds(st + 1, 16)][0]
        base0 = (e_lo // 8) * 8
        nch = (e_hi - base0 + CH - 1) // CH

        def _cb(k):
            return jnp.minimum(base0 + k * CH, np_edges - CH)

        def sync_j(k, b):
            pltpu.sync_copy(idxj_hbm.at[pl.ds(_cb(k), CH)], ij[b])

        def pre_j(k, b):
            pltpu.async_copy(idxj_hbm.at[pl.ds(_cb(k), CH)],
                             ij[b], jsem[b])

        def wait_j(b):
            pltpu.make_async_copy(idxj_hbm.at[pl.ds(0, CH)],
                                  ij[b], jsem[b]).wait()

        def issue_data(k, b):
            cb = _cb(k)
            pltpu.async_copy(xcat_hbm.at[ij[b]], xg[b], dsem[b])
            pltpu.async_copy(gcat_hbm.at[pl.ds(cb, CH)], gg[b], dsem[b])
            pltpu.async_copy(pd_hbm.at[pl.ds(cb, CH)], pdb[b], dsem[b])

        def wait_data(b):
            pltpu.make_async_copy(xcat_hbm.at[ij[b]], xg[b], dsem[b]).wait()
            pltpu.make_async_copy(gcat_hbm.at[pl.ds(0, CH)],
                                  gg[b], dsem[b]).wait()
            pltpu.make_async_copy(pd_hbm.at[pl.ds(0, CH)],
                                  pdb[b], dsem[b]).wait()

        def compute(k, b):
            cb_raw = base0 + k * CH
            cb = _cb(k)
            xg_v = xg[b]
            gpd_v = gg[b]
            pd_v = pdb[b]
            lo = jnp.maximum(e_lo, cb_raw) - cb
            hi = jnp.minimum(e_hi, cb_raw + CH) - cb
            cab0 = pd_v[jnp.minimum(lo, CH - 1)][3].astype(jnp.int32) * ACC_W

            for grp in ((0, 1, 2), (3, 4, 5), (6, 7)):
                offs16 = tuple(g * 16 for g in grp)
                UB = len(grp)
                NA = 9 * UB

                def edge(e, c4):
                    acc = c4[:NA]
                    cab = c4[NA]
                    pdr = pd_v[e]
                    isnew = pdr[4] > 0.5

                    @pl.when(isnew)
                    def _flush():
                        for u in range(UB):
                            for s9 in range(9):
                                plsc.addupdate(
                                    acc_v.at[pl.ds(cab + s9 * D + offs16[u],
                                                   16)],
                                    acc[u * 9 + s9])

                    ab = pdr[3].astype(jnp.int32) * ACC_W
                    cab_n = jnp.where(isnew, ab, cab)
                    sc = (pdr[0], pdr[1], pdr[2],
                          pdr[8], pdr[9], pdr[10], pdr[11], pdr[12])
                    hmask = jnp.int32(-65536)

                    def _lo(word):
                        return lax.bitcast_convert_type(word << 16,
                                                        jnp.float32)

                    def _hi(word):
                        return lax.bitcast_convert_type(word & hmask,
                                                        jnp.float32)

                    na = []
                    for u in range(UB):
                        oo = offs16[u]
                        half = u * 9
                        w_xsp = xg_v[e, pl.ds(oo, 16)]
                        w_gsp = gpd_v[e, pl.ds(oo, 16)]
                        w_xd = xg_v[e, pl.ds(D + oo, 16)]
                        w_gd = gpd_v[e, pl.ds(D + oo, 16)]
                        hs = _lo(w_xsp) * _lo(w_gsp)
                        hp = _hi(w_xsp) * _hi(w_gsp)
                        hd = _lo(w_xd) * _lo(w_gd)
                        na.append(jnp.where(isnew, zv, acc[half]) + hs)
                        for c3i in range(3):
                            na.append(jnp.where(isnew, zv, acc[half + 1 + c3i])
                                      + sc[c3i] * hp)
                        for c5i in range(5):
                            na.append(jnp.where(isnew, zv, acc[half + 4 + c5i])
                                      + sc[3 + c5i] * hd)
                    return tuple(na) + (cab_n,)

                fa = lax.fori_loop(
                    lo, hi, edge,
                    (zv,) * NA + (cab0,))
                for u in range(UB):
                    for s9 in range(9):
                        plsc.addupdate(
                            acc_v.at[pl.ds(fa[NA] + s9 * D + offs16[u], 16)],
                            fa[u * 9 + s9])

        @pl.when(nch > 0)
        def _prologue():
            sync_j(0, 0)
            issue_data(0, 0)

            @pl.when(nch > 1)
            def _pro2():
                sync_j(1, 1)

        def pair(q, c3):
            k0 = 2 * q
            k1 = k0 + 1

            @pl.when(k1 < nch)
            def _iss1():
                @pl.when(k1 >= 2)
                def _wj1():
                    wait_j(1)
                issue_data(k1, 1)

            wait_data(0)

            @pl.when(k0 + 2 < nch)
            def _pj0():
                pre_j(k0 + 2, 0)

            compute(k0, 0)

            @pl.when(k1 < nch)
            def _phase_b():
                wait_data(1)

                @pl.when(k1 + 2 < nch)
                def _pj1():
                    pre_j(k1 + 2, 1)

                @pl.when(k1 + 1 < nch)
                def _iss0():
                    wait_j(0)
                    issue_data(k1 + 1, 0)

                compute(k1, 1)
            return c3
        lax.fori_loop(0, (nch + 1) // 2, pair, 0)
        pltpu.sync_copy(
            acc_v, out_hbm.at[pl.ds(st * T_NODES * ACC_W, T_NODES * ACC_W)])
        return carry
    lax.fori_loop(0, per_w, subtile, 0)


def _final_body(acc_ref, xx_ref, ppt_ref, pdt_ref,
                ow1, ob1, ow2, ob2, owf, obf, out_ref):
    acc = acc_ref[...]
    ppt = ppt_ref[...]
    pdt = pdt_ref[...]
    s = xx_ref[...] + acc[:, 0:D]
    tot = s
    for c in range(3):
        pc = acc[:, D + c * D: D + (c + 1) * D]
        pr = jnp.dot(pc, ppt, preferred_element_type=jnp.float32)
        tot = tot + pr[:, :D] * pr[:, D:]
    for c in range(5):
        dc = acc[:, 4 * D + c * D: 4 * D + (c + 1) * D]
        dr = jnp.dot(dc, pdt, preferred_element_type=jnp.float32)
        tot = tot + dr[:, :D] * dr[:, D:]
    out_ref[...] = _resmlp(tot, ow1[...], ob1[...], ow2[...], ob2[...],
                           owf[...], obf[...])


def _mlp_args(p):
    return (p['W1'].T, p['b1'].reshape(1, D), p['W2'].T, p['b2'].reshape(1, D),
            p['Wf'].T, p['bf'].reshape(1, D))


def kernel(x, rbf, pij, dij, rs_W, rp_W, rd_W, proj_p, proj_d,
           mlp_x, mlp_s, mlp_p, mlp_d, mlp_o, idx_i, idx_j):
    n = x.shape[0]
    p = idx_i.shape[0]

    nt = -(-n // T_NODES)
    nt = -(-nt // NW) * NW
    n_pad = nt * T_NODES
    n_pad = -(-n_pad // NBLK) * NBLK
    nt = n_pad // T_NODES
    p_pad = p

    xq = jnp.pad(x, ((0, n_pad - n), (0, 0)))
    li_f = (idx_i % T_NODES).astype(jnp.float32)[:, None]
    new_f = jnp.concatenate(
        [jnp.ones((1,), jnp.float32),
         (jnp.diff(idx_i) != 0).astype(jnp.float32)])[:, None]
    pdq = jnp.concatenate(
        [pij, li_f, new_f, jnp.zeros((p, 3), jnp.float32),
         dij, jnp.zeros((p, 3), jnp.float32)], axis=1)
    del li_f, new_f
    bounds = jnp.arange(nt + 1, dtype=jnp.int32) * T_NODES
    offs = jnp.searchsorted(idx_i, bounds, side='left').astype(jnp.int32)
    noffs = -(-(nt + 1 + 16) // 16) * 16
    offs = jnp.pad(offs, (0, noffs - (nt + 1)), constant_values=p)

    wg = jnp.concatenate([rs_W, rp_W, rd_W], axis=0).T

    mlp_in = (_mlp_args(mlp_x) + _mlp_args(mlp_s) + _mlp_args(mlp_p)
              + _mlp_args(mlp_d))
    grid_n = n_pad // NBLK
    row_spec = pl.BlockSpec((NBLK, D), lambda i: (i, 0))
    full = lambda shp: pl.BlockSpec(shp, lambda i: tuple(0 for _ in shp))
    xx, xcat = pl.pallas_call(
        _nodes_body,
        grid=(grid_n,),
        in_specs=[row_spec] + [full(a.shape) for a in mlp_in],
        out_specs=[row_spec, pl.BlockSpec((NBLK, 2 * D), lambda i: (i, 0))],
        out_shape=[jax.ShapeDtypeStruct((n_pad, D), jnp.float32),
                   jax.ShapeDtypeStruct((n_pad, 2 * D), jnp.int32)],
    )(xq, *mlp_in)

    grid_p = p_pad // GBLK
    gcat = pl.pallas_call(
        _gcat_body,
        grid=(grid_p,),
        in_specs=[pl.BlockSpec((GBLK, R), lambda i: (i, 0)), full(wg.shape)],
        out_specs=pl.BlockSpec((GBLK, 2 * D), lambda i: (i, 0)),
        out_shape=jax.ShapeDtypeStruct((p_pad, 2 * D), jnp.int32),
    )(rbf, wg)

    mesh = plsc.VectorSubcoreMesh(core_axis_name="c", subcore_axis_name="s")
    acc_flat = pl.kernel(
        functools.partial(_sc_body, n_pad, p_pad, nt),
        mesh=mesh,
        out_type=jax.ShapeDtypeStruct((n_pad * ACC_W,), jnp.float32),
        scratch_types=[
            pltpu.VMEM((noffs,), jnp.int32),
            pltpu.VMEM((CH,), jnp.int32),
            pltpu.VMEM((CH,), jnp.int32),
            pltpu.VMEM((CH, 2 * D), jnp.int32),
            pltpu.VMEM((CH, 2 * D), jnp.int32),
            pltpu.VMEM((CH, 2 * D), jnp.int32),
            pltpu.VMEM((CH, 2 * D), jnp.int32),
            pltpu.VMEM((CH, 16), jnp.float32),
            pltpu.VMEM((CH, 16), jnp.float32),
            pltpu.VMEM((T_NODES * ACC_W,), jnp.float32),
            pltpu.SemaphoreType.DMA,
            pltpu.SemaphoreType.DMA,
            pltpu.SemaphoreType.DMA,
            pltpu.SemaphoreType.DMA,
        ],
    )(offs, idx_j, xcat, gcat, pdq)
    acc = acc_flat.reshape(n_pad, ACC_W)

    pptq = proj_p.T
    pdtq = proj_d.T
    fin_in = (_mlp_args(mlp_o))
    out = pl.pallas_call(
        _final_body,
        grid=(grid_n,),
        in_specs=[pl.BlockSpec((NBLK, ACC_W), lambda i: (i, 0)), row_spec,
                  full(pptq.shape), full(pdtq.shape)]
                 + [full(a.shape) for a in fin_in],
        out_specs=row_spec,
        out_shape=jax.ShapeDtypeStruct((n_pad, D), jnp.float32),
    )(acc, xx, pptq, pdtq, *fin_in)

    return out[:n]


# speedup vs baseline: 1.1070x; 1.1070x over previous
import functools

import jax
import jax.numpy as jnp
import numpy as np
from jax import lax
from jax.experimental import pallas as pl
from jax.experimental.pallas import tpu as pltpu
from jax.experimental.pallas import tpu_sc as plsc

D = 128
R = 16
T_NODES = 32
CH = 72
NC = 2
NS = 16
NW = NC * NS
ACC_W = 9 * D
NBLK = 256
GBLK = 1000


def _swish(v):
    return v * jax.nn.sigmoid(v)


def _bf16_bits(y):
    u = lax.bitcast_convert_type(y, jnp.int32)
    return (u + 0x7FFF + ((u >> 16) & 1)) >> 16


def _pack_pairs(a, b, c):
    w1 = (_bf16_bits(a) & 0xFFFF) | (_bf16_bits(b) << 16)
    w2 = _bf16_bits(c) & 0xFFFF
    return jnp.concatenate([w1, w2], axis=1)


def _resmlp(v, w1t, b1, w2t, b2, wft, bf):
    y = _swish(v)
    y = jnp.dot(y, w1t, preferred_element_type=jnp.float32) + b1
    y = _swish(y)
    y = jnp.dot(y, w2t, preferred_element_type=jnp.float32) + b2
    v = v + y
    return jnp.dot(_swish(v), wft, preferred_element_type=jnp.float32) + bf


def _nodes_body(x_ref, *refs):
    (xw1, xb1, xw2, xb2, xwf, xbf,
     sw1, sb1, sw2, sb2, swf, sbf,
     pw1, pb1, pw2, pb2, pwf, pbf,
     dw1, db1, dw2, db2, dwf, dbf,
     xx_ref, xcat_ref) = refs
    xb = x_ref[...]
    xx = _resmlp(xb, xw1[...], xb1[...], xw2[...], xb2[...], xwf[...], xbf[...])
    xs = _resmlp(xb, sw1[...], sb1[...], sw2[...], sb2[...], swf[...], sbf[...])
    xp = _resmlp(xb, pw1[...], pb1[...], pw2[...], pb2[...], pwf[...], pbf[...])
    xd = _resmlp(xb, dw1[...], db1[...], dw2[...], db2[...], dwf[...], dbf[...])
    xx_ref[...] = xx
    xcat_ref[...] = _pack_pairs(xs, xp, xd)


def _gcat_body(rbf_ref, wg_ref, gcat_ref):
    g = jnp.dot(rbf_ref[...], wg_ref[...],
                preferred_element_type=jnp.float32)
    gcat_ref[...] = _pack_pairs(g[:, :D], g[:, D:2 * D], g[:, 2 * D:])


def _sc_body(n_pad, p_pad, nt,
             offs_hbm, idxj_hbm, xcat_hbm, gcat_hbm, pd_hbm, out_hbm,
             offs_v, ij0, ij1, xg0, xg1, gg0, gg1, pd0, pd1, acc_v,
             dsem0, dsem1, jsem0, jsem1):
    del n_pad
    np_edges = p_pad
    w = lax.axis_index("s") * NC + lax.axis_index("c")
    per_w = nt // NW
    pltpu.sync_copy(offs_hbm, offs_v)
    zv = jnp.zeros((16,), jnp.float32)
    ij = (ij0, ij1)
    xg = (xg0, xg1)
    gg = (gg0, gg1)
    pdb = (pd0, pd1)
    dsem = (dsem0, dsem1)
    jsem = (jsem0, jsem1)

    def subtile(t, carry):
        st = t * NW + w

        def zbody(z, c2):
            acc_v[pl.ds(z * 16, 16)] = zv
            return c2
        lax.fori_loop(0, (T_NODES * ACC_W) // 16, zbody, 0)

        e_lo = offs_v[pl.ds(st, 16)][0]
        e_hi = offs_v[pl.ds(st + 1, 16)][0]
        base0 = (e_lo // 8) * 8
        nch = (e_hi - base0 + CH - 1) // CH

        def _cb(k):
            return jnp.minimum(base0 + k * CH, np_edges - CH)

        def sync_j(k, b):
            pltpu.sync_copy(idxj_hbm.at[pl.ds(_cb(k), CH)], ij[b])

        def pre_j(k, b):
            pltpu.async_copy(idxj_hbm.at[pl.ds(_cb(k), CH)],
                             ij[b], jsem[b])

        def wait_j(b):
            pltpu.make_async_copy(idxj_hbm.at[pl.ds(0, CH)],
                                  ij[b], jsem[b]).wait()

        def issue_data(k, b):
            cb = _cb(k)
            pltpu.async_copy(xcat_hbm.at[ij[b]], xg[b], dsem[b])
            pltpu.async_copy(gcat_hbm.at[pl.ds(cb, CH)], gg[b], dsem[b])
            pltpu.async_copy(pd_hbm.at[pl.ds(cb, CH)], pdb[b], dsem[b])

        def wait_data(b):
            pltpu.make_async_copy(xcat_hbm.at[ij[b]], xg[b], dsem[b]).wait()
            pltpu.make_async_copy(gcat_hbm.at[pl.ds(0, CH)],
                                  gg[b], dsem[b]).wait()
            pltpu.make_async_copy(pd_hbm.at[pl.ds(0, CH)],
                                  pdb[b], dsem[b]).wait()

        def compute(k, b):
            cb_raw = base0 + k * CH
            cb = _cb(k)
            xg_v = xg[b]
            gpd_v = gg[b]
            pd_v = pdb[b]
            lo = jnp.maximum(e_lo, cb_raw) - cb
            hi = jnp.minimum(e_hi, cb_raw + CH) - cb
            cab0 = pd_v[jnp.minimum(lo, CH - 1)][3].astype(jnp.int32) * ACC_W

            for grp in ((0, 1), (2, 3), (4, 5), (6, 7)):
                offs16 = tuple(g * 16 for g in grp)
                UB = len(grp)
                NA = 9 * UB

                def edge(e, c4):
                    acc = c4[:NA]
                    cab = c4[NA]
                    pdr = pd_v[e]
                    isnew = pdr[4] > 0.5

                    @pl.when(isnew)
                    def _flush():
                        for u in range(UB):
                            for s9 in range(9):
                                plsc.addupdate(
                                    acc_v.at[pl.ds(cab + s9 * D + offs16[u],
                                                   16)],
                                    acc[u * 9 + s9])

                    ab = pdr[3].astype(jnp.int32) * ACC_W
                    cab_n = jnp.where(isnew, ab, cab)
                    sc = (pdr[0], pdr[1], pdr[2],
                          pdr[8], pdr[9], pdr[10], pdr[11], pdr[12])
                    hmask = jnp.int32(-65536)

                    def _lo(word):
                        return lax.bitcast_convert_type(word << 16,
                                                        jnp.float32)

                    def _hi(word):
                        return lax.bitcast_convert_type(word & hmask,
                                                        jnp.float32)

                    na = []
                    for u in range(UB):
                        oo = offs16[u]
                        half = u * 9
                        w_xsp = xg_v[e, pl.ds(oo, 16)]
                        w_gsp = gpd_v[e, pl.ds(oo, 16)]
                        w_xd = xg_v[e, pl.ds(D + oo, 16)]
                        w_gd = gpd_v[e, pl.ds(D + oo, 16)]
                        hs = _lo(w_xsp) * _lo(w_gsp)
                        hp = _hi(w_xsp) * _hi(w_gsp)
                        hd = _lo(w_xd) * _lo(w_gd)
                        na.append(jnp.where(isnew, zv, acc[half]) + hs)
                        for c3i in range(3):
                            na.append(jnp.where(isnew, zv, acc[half + 1 + c3i])
                                      + sc[c3i] * hp)
                        for c5i in range(5):
                            na.append(jnp.where(isnew, zv, acc[half + 4 + c5i])
                                      + sc[3 + c5i] * hd)
                    return tuple(na) + (cab_n,)

                fa = lax.fori_loop(
                    lo, hi, edge,
                    (zv,) * NA + (cab0,))
                for u in range(UB):
                    for s9 in range(9):
                        plsc.addupdate(
                            acc_v.at[pl.ds(fa[NA] + s9 * D + offs16[u], 16)],
                            fa[u * 9 + s9])

        @pl.when(nch > 0)
        def _prologue():
            sync_j(0, 0)
            issue_data(0, 0)

            @pl.when(nch > 1)
            def _pro2():
                sync_j(1, 1)

        def pair(q, c3):
            k0 = 2 * q
            k1 = k0 + 1

            @pl.when(k1 < nch)
            def _iss1():
                @pl.when(k1 >= 2)
                def _wj1():
                    wait_j(1)
                issue_data(k1, 1)

            wait_data(0)

            @pl.when(k0 + 2 < nch)
            def _pj0():
                pre_j(k0 + 2, 0)

            compute(k0, 0)

            @pl.when(k1 < nch)
            def _phase_b():
                wait_data(1)

                @pl.when(k1 + 2 < nch)
                def _pj1():
                    pre_j(k1 + 2, 1)

                @pl.when(k1 + 1 < nch)
                def _iss0():
                    wait_j(0)
                    issue_data(k1 + 1, 0)

                compute(k1, 1)
            return c3
        lax.fori_loop(0, (nch + 1) // 2, pair, 0)
        pltpu.sync_copy(
            acc_v, out_hbm.at[pl.ds(st * T_NODES * ACC_W, T_NODES * ACC_W)])
        return carry
    lax.fori_loop(0, per_w, subtile, 0)


def _final_body(acc_ref, xx_ref, ppt_ref, pdt_ref,
                ow1, ob1, ow2, ob2, owf, obf, out_ref):
    acc = acc_ref[...]
    ppt = ppt_ref[...]
    pdt = pdt_ref[...]
    s = xx_ref[...] + acc[:, 0:D]
    tot = s
    for c in range(3):
        pc = acc[:, D + c * D: D + (c + 1) * D]
        pr = jnp.dot(pc, ppt, preferred_element_type=jnp.float32)
        tot = tot + pr[:, :D] * pr[:, D:]
    for c in range(5):
        dc = acc[:, 4 * D + c * D: 4 * D + (c + 1) * D]
        dr = jnp.dot(dc, pdt, preferred_element_type=jnp.float32)
        tot = tot + dr[:, :D] * dr[:, D:]
    out_ref[...] = _resmlp(tot, ow1[...], ob1[...], ow2[...], ob2[...],
                           owf[...], obf[...])


def _mlp_args(p):
    return (p['W1'].T, p['b1'].reshape(1, D), p['W2'].T, p['b2'].reshape(1, D),
            p['Wf'].T, p['bf'].reshape(1, D))


def kernel(x, rbf, pij, dij, rs_W, rp_W, rd_W, proj_p, proj_d,
           mlp_x, mlp_s, mlp_p, mlp_d, mlp_o, idx_i, idx_j):
    n = x.shape[0]
    p = idx_i.shape[0]

    nt = -(-n // T_NODES)
    nt = -(-nt // NW) * NW
    n_pad = nt * T_NODES
    n_pad = -(-n_pad // NBLK) * NBLK
    nt = n_pad // T_NODES
    p_pad = p

    xq = jnp.pad(x, ((0, n_pad - n), (0, 0)))
    li_f = (idx_i % T_NODES).astype(jnp.float32)[:, None]
    new_f = jnp.concatenate(
        [jnp.ones((1,), jnp.float32),
         (jnp.diff(idx_i) != 0).astype(jnp.float32)])[:, None]
    pdq = jnp.concatenate(
        [pij, li_f, new_f, jnp.zeros((p, 3), jnp.float32),
         dij, jnp.zeros((p, 3), jnp.float32)], axis=1)
    del li_f, new_f
    bounds = jnp.arange(nt + 1, dtype=jnp.int32) * T_NODES
    offs = jnp.searchsorted(idx_i, bounds, side='left').astype(jnp.int32)
    noffs = -(-(nt + 1 + 16) // 16) * 16
    offs = jnp.pad(offs, (0, noffs - (nt + 1)), constant_values=p)

    wg = jnp.concatenate([rs_W, rp_W, rd_W], axis=0).T

    mlp_in = (_mlp_args(mlp_x) + _mlp_args(mlp_s) + _mlp_args(mlp_p)
              + _mlp_args(mlp_d))
    grid_n = n_pad // NBLK
    row_spec = pl.BlockSpec((NBLK, D), lambda i: (i, 0))
    full = lambda shp: pl.BlockSpec(shp, lambda i: tuple(0 for _ in shp))
    xx, xcat = pl.pallas_call(
        _nodes_body,
        grid=(grid_n,),
        in_specs=[row_spec] + [full(a.shape) for a in mlp_in],
        out_specs=[row_spec, pl.BlockSpec((NBLK, 2 * D), lambda i: (i, 0))],
        out_shape=[jax.ShapeDtypeStruct((n_pad, D), jnp.float32),
                   jax.ShapeDtypeStruct((n_pad, 2 * D), jnp.int32)],
    )(xq, *mlp_in)

    grid_p = p_pad // GBLK
    gcat = pl.pallas_call(
        _gcat_body,
        grid=(grid_p,),
        in_specs=[pl.BlockSpec((GBLK, R), lambda i: (i, 0)), full(wg.shape)],
        out_specs=pl.BlockSpec((GBLK, 2 * D), lambda i: (i, 0)),
        out_shape=jax.ShapeDtypeStruct((p_pad, 2 * D), jnp.int32),
    )(rbf, wg)

    mesh = plsc.VectorSubcoreMesh(core_axis_name="c", subcore_axis_name="s")
    acc_flat = pl.kernel(
        functools.partial(_sc_body, n_pad, p_pad, nt),
        mesh=mesh,
        out_type=jax.ShapeDtypeStruct((n_pad * ACC_W,), jnp.float32),
        scratch_types=[
            pltpu.VMEM((noffs,), jnp.int32),
            pltpu.VMEM((CH,), jnp.int32),
            pltpu.VMEM((CH,), jnp.int32),
            pltpu.VMEM((CH, 2 * D), jnp.int32),
            pltpu.VMEM((CH, 2 * D), jnp.int32),
            pltpu.VMEM((CH, 2 * D), jnp.int32),
            pltpu.VMEM((CH, 2 * D), jnp.int32),
            pltpu.VMEM((CH, 16), jnp.float32),
            pltpu.VMEM((CH, 16), jnp.float32),
            pltpu.VMEM((T_NODES * ACC_W,), jnp.float32),
            pltpu.SemaphoreType.DMA,
            pltpu.SemaphoreType.DMA,
            pltpu.SemaphoreType.DMA,
            pltpu.SemaphoreType.DMA,
        ],
    )(offs, idx_j, xcat, gcat, pdq)
    acc = acc_flat.reshape(n_pad, ACC_W)

    pptq = proj_p.T
    pdtq = proj_d.T
    fin_in = (_mlp_args(mlp_o))
    out = pl.pallas_call(
        _final_body,
        grid=(grid_n,),
        in_specs=[pl.BlockSpec((NBLK, ACC_W), lambda i: (i, 0)), row_spec,
                  full(pptq.shape), full(pdtq.shape)]
                 + [full(a.shape) for a in fin_in],
        out_specs=row_spec,
        out_shape=jax.ShapeDtypeStruct((n_pad, D), jnp.float32),
    )(acc, xx, pptq, pdtq, *fin_in)

    return out[:n]


# speedup vs baseline: 1.1092x; 1.0020x over previous
import functools

import jax
import jax.numpy as jnp
import numpy as np
from jax import lax
from jax.experimental import pallas as pl
from jax.experimental.pallas import tpu as pltpu
from jax.experimental.pallas import tpu_sc as plsc

D = 128
R = 16
T_NODES = 32
CH = 64
NC = 2
NS = 16
NW = NC * NS
ACC_W = 9 * D
NBLK = 256
GBLK = 1000


def _swish(v):
    return v * jax.nn.sigmoid(v)


def _bf16_bits(y):
    u = lax.bitcast_convert_type(y, jnp.int32)
    return (u + 0x7FFF + ((u >> 16) & 1)) >> 16


def _pack_pairs(a, b, c):
    w1 = (_bf16_bits(a) & 0xFFFF) | (_bf16_bits(b) << 16)
    w2 = _bf16_bits(c) & 0xFFFF
    return jnp.concatenate([w1, w2], axis=1)


def _resmlp(v, w1t, b1, w2t, b2, wft, bf):
    y = _swish(v)
    y = jnp.dot(y, w1t, preferred_element_type=jnp.float32) + b1
    y = _swish(y)
    y = jnp.dot(y, w2t, preferred_element_type=jnp.float32) + b2
    v = v + y
    return jnp.dot(_swish(v), wft, preferred_element_type=jnp.float32) + bf


def _nodes_body(x_ref, *refs):
    (xw1, xb1, xw2, xb2, xwf, xbf,
     sw1, sb1, sw2, sb2, swf, sbf,
     pw1, pb1, pw2, pb2, pwf, pbf,
     dw1, db1, dw2, db2, dwf, dbf,
     xx_ref, xcat_ref) = refs
    xb = x_ref[...]
    xx = _resmlp(xb, xw1[...], xb1[...], xw2[...], xb2[...], xwf[...], xbf[...])
    xs = _resmlp(xb, sw1[...], sb1[...], sw2[...], sb2[...], swf[...], sbf[...])
    xp = _resmlp(xb, pw1[...], pb1[...], pw2[...], pb2[...], pwf[...], pbf[...])
    xd = _resmlp(xb, dw1[...], db1[...], dw2[...], db2[...], dwf[...], dbf[...])
    xx_ref[...] = xx
    xcat_ref[...] = _pack_pairs(xs, xp, xd)


def _gcat_body(rbf_ref, wg_ref, gcat_ref):
    g = jnp.dot(rbf_ref[...], wg_ref[...],
                preferred_element_type=jnp.float32)
    gcat_ref[...] = _pack_pairs(g[:, :D], g[:, D:2 * D], g[:, 2 * D:])


def _sc_body(n_pad, p_pad, nt,
             offs_hbm, idxj_hbm, xcat_hbm, gcat_hbm, pd_hbm, out_hbm,
             offs_v, ij0, ij1, xg0, xg1, gg0, gg1, pd0, pd1, acc_v,
             dsem0, dsem1, jsem0, jsem1):
    del n_pad
    np_edges = p_pad
    w = lax.axis_index("s") * NC + lax.axis_index("c")
    per_w = nt // NW
    pltpu.sync_copy(offs_hbm, offs_v)
    zv = jnp.zeros((16,), jnp.float32)
    ij = (ij0, ij1)
    xg = (xg0, xg1)
    gg = (gg0, gg1)
    pdb = (pd0, pd1)
    dsem = (dsem0, dsem1)
    jsem = (jsem0, jsem1)

    def subtile(t, carry):
        st = t * NW + w

        def zbody(z, c2):
            acc_v[pl.ds(z * 16, 16)] = zv
            return c2
        lax.fori_loop(0, (T_NODES * ACC_W) // 16, zbody, 0)

        e_lo = offs_v[pl.ds(st, 16)][0]
        e_hi = offs_v[pl.ds(st + 1, 16)][0]
        base0 = (e_lo // 8) * 8
        nch = (e_hi - base0 + CH - 1) // CH

        def _cb(k):
            return jnp.minimum(base0 + k * CH, np_edges - CH)

        def sync_j(k, b):
            pltpu.sync_copy(idxj_hbm.at[pl.ds(_cb(k), CH)], ij[b])

        def pre_j(k, b):
            pltpu.async_copy(idxj_hbm.at[pl.ds(_cb(k), CH)],
                             ij[b], jsem[b])

        def wait_j(b):
            pltpu.make_async_copy(idxj_hbm.at[pl.ds(0, CH)],
                                  ij[b], jsem[b]).wait()

        def issue_data(k, b):
            cb = _cb(k)
            pltpu.async_copy(xcat_hbm.at[ij[b]], xg[b], dsem[b])
            pltpu.async_copy(gcat_hbm.at[pl.ds(cb, CH)], gg[b], dsem[b])
            pltpu.async_copy(pd_hbm.at[pl.ds(cb, CH)], pdb[b], dsem[b])

        def wait_data(b):
            pltpu.make_async_copy(xcat_hbm.at[ij[b]], xg[b], dsem[b]).wait()
            pltpu.make_async_copy(gcat_hbm.at[pl.ds(0, CH)],
                                  gg[b], dsem[b]).wait()
            pltpu.make_async_copy(pd_hbm.at[pl.ds(0, CH)],
                                  pdb[b], dsem[b]).wait()

        def compute(k, b):
            cb_raw = base0 + k * CH
            cb = _cb(k)
            xg_v = xg[b]
            gpd_v = gg[b]
            pd_v = pdb[b]
            lo = jnp.maximum(e_lo, cb_raw) - cb
            hi = jnp.minimum(e_hi, cb_raw + CH) - cb
            cab0 = pd_v[jnp.minimum(lo, CH - 1)][3].astype(jnp.int32) * ACC_W

            for grp in ((0, 1), (2, 3), (4, 5), (6, 7)):
                offs16 = tuple(g * 16 for g in grp)
                UB = len(grp)
                NA = 9 * UB

                def edge(e, c4):
                    acc = c4[:NA]
                    cab = c4[NA]
                    pdr = pd_v[e]
                    isnew = pdr[4] > 0.5

                    @pl.when(isnew)
                    def _flush():
                        for u in range(UB):
                            for s9 in range(9):
                                plsc.addupdate(
                                    acc_v.at[pl.ds(cab + s9 * D + offs16[u],
                                                   16)],
                                    acc[u * 9 + s9])

                    ab = pdr[3].astype(jnp.int32) * ACC_W
                    cab_n = jnp.where(isnew, ab, cab)
                    sc = (pdr[0], pdr[1], pdr[2],
                          pdr[8], pdr[9], pdr[10], pdr[11], pdr[12])
                    hmask = jnp.int32(-65536)

                    def _lo(word):
                        return lax.bitcast_convert_type(word << 16,
                                                        jnp.float32)

                    def _hi(word):
                        return lax.bitcast_convert_type(word & hmask,
                                                        jnp.float32)

                    na = []
                    for u in range(UB):
                        oo = offs16[u]
                        half = u * 9
                        w_xsp = xg_v[e, pl.ds(oo, 16)]
                        w_gsp = gpd_v[e, pl.ds(oo, 16)]
                        w_xd = xg_v[e, pl.ds(D + oo, 16)]
                        w_gd = gpd_v[e, pl.ds(D + oo, 16)]
                        hs = _lo(w_xsp) * _lo(w_gsp)
                        hp = _hi(w_xsp) * _hi(w_gsp)
                        hd = _lo(w_xd) * _lo(w_gd)
                        na.append(jnp.where(isnew, zv, acc[half]) + hs)
                        for c3i in range(3):
                            na.append(jnp.where(isnew, zv, acc[half + 1 + c3i])
                                      + sc[c3i] * hp)
                        for c5i in range(5):
                            na.append(jnp.where(isnew, zv, acc[half + 4 + c5i])
                                      + sc[3 + c5i] * hd)
                    return tuple(na) + (cab_n,)

                fa = lax.fori_loop(
                    lo, hi, edge,
                    (zv,) * NA + (cab0,))
                for u in range(UB):
                    for s9 in range(9):
                        plsc.addupdate(
                            acc_v.at[pl.ds(fa[NA] + s9 * D + offs16[u], 16)],
                            fa[u * 9 + s9])

        @pl.when(nch > 0)
        def _prologue():
            pre_j(0, 0)

            @pl.when(nch > 1)
            def _pro2():
                pre_j(1, 1)

            wait_j(0)
            issue_data(0, 0)

        def pair(q, c3):
            k0 = 2 * q
            k1 = k0 + 1

            @pl.when(k1 < nch)
            def _iss1():
                wait_j(1)
                issue_data(k1, 1)

            wait_data(0)

            @pl.when(k0 + 2 < nch)
            def _pj0():
                pre_j(k0 + 2, 0)

            compute(k0, 0)

            @pl.when(k1 < nch)
            def _phase_b():
                wait_data(1)

                @pl.when(k1 + 2 < nch)
                def _pj1():
                    pre_j(k1 + 2, 1)

                @pl.when(k1 + 1 < nch)
                def _iss0():
                    wait_j(0)
                    issue_data(k1 + 1, 0)

                compute(k1, 1)
            return c3
        lax.fori_loop(0, (nch + 1) // 2, pair, 0)
        pltpu.sync_copy(
            acc_v, out_hbm.at[pl.ds(st * T_NODES * ACC_W, T_NODES * ACC_W)])
        return carry
    lax.fori_loop(0, per_w, subtile, 0)


def _final_body(acc_ref, xx_ref, ppt_ref, pdt_ref,
                ow1, ob1, ow2, ob2, owf, obf, out_ref):
    acc = acc_ref[...]
    ppt = ppt_ref[...]
    pdt = pdt_ref[...]
    s = xx_ref[...] + acc[:, 0:D]
    tot = s
    for c in range(3):
        pc = acc[:, D + c * D: D + (c + 1) * D]
        pr = jnp.dot(pc, ppt, preferred_element_type=jnp.float32)
        tot = tot + pr[:, :D] * pr[:, D:]
    for c in range(5):
        dc = acc[:, 4 * D + c * D: 4 * D + (c + 1) * D]
        dr = jnp.dot(dc, pdt, preferred_element_type=jnp.float32)
        tot = tot + dr[:, :D] * dr[:, D:]
    out_ref[...] = _resmlp(tot, ow1[...], ob1[...], ow2[...], ob2[...],
                           owf[...], obf[...])


def _mlp_args(p):
    return (p['W1'].T, p['b1'].reshape(1, D), p['W2'].T, p['b2'].reshape(1, D),
            p['Wf'].T, p['bf'].reshape(1, D))


def kernel(x, rbf, pij, dij, rs_W, rp_W, rd_W, proj_p, proj_d,
           mlp_x, mlp_s, mlp_p, mlp_d, mlp_o, idx_i, idx_j):
    n = x.shape[0]
    p = idx_i.shape[0]

    nt = -(-n // T_NODES)
    nt = -(-nt // NW) * NW
    n_pad = nt * T_NODES
    n_pad = -(-n_pad // NBLK) * NBLK
    nt = n_pad // T_NODES
    p_pad = p

    xq = jnp.pad(x, ((0, n_pad - n), (0, 0)))
    li_f = (idx_i % T_NODES).astype(jnp.float32)[:, None]
    new_f = jnp.concatenate(
        [jnp.ones((1,), jnp.float32),
         (jnp.diff(idx_i) != 0).astype(jnp.float32)])[:, None]
    pdq = jnp.concatenate(
        [pij, li_f, new_f, jnp.zeros((p, 3), jnp.float32),
         dij, jnp.zeros((p, 3), jnp.float32)], axis=1)
    del li_f, new_f
    bounds = jnp.arange(nt + 1, dtype=jnp.int32) * T_NODES
    offs = jnp.searchsorted(idx_i, bounds, side='left').astype(jnp.int32)
    noffs = -(-(nt + 1 + 16) // 16) * 16
    offs = jnp.pad(offs, (0, noffs - (nt + 1)), constant_values=p)

    wg = jnp.concatenate([rs_W, rp_W, rd_W], axis=0).T

    mlp_in = (_mlp_args(mlp_x) + _mlp_args(mlp_s) + _mlp_args(mlp_p)
              + _mlp_args(mlp_d))
    grid_n = n_pad // NBLK
    row_spec = pl.BlockSpec((NBLK, D), lambda i: (i, 0))
    full = lambda shp: pl.BlockSpec(shp, lambda i: tuple(0 for _ in shp))
    xx, xcat = pl.pallas_call(
        _nodes_body,
        grid=(grid_n,),
        in_specs=[row_spec] + [full(a.shape) for a in mlp_in],
        out_specs=[row_spec, pl.BlockSpec((NBLK, 2 * D), lambda i: (i, 0))],
        out_shape=[jax.ShapeDtypeStruct((n_pad, D), jnp.float32),
                   jax.ShapeDtypeStruct((n_pad, 2 * D), jnp.int32)],
    )(xq, *mlp_in)

    grid_p = p_pad // GBLK
    gcat = pl.pallas_call(
        _gcat_body,
        grid=(grid_p,),
        in_specs=[pl.BlockSpec((GBLK, R), lambda i: (i, 0)), full(wg.shape)],
        out_specs=pl.BlockSpec((GBLK, 2 * D), lambda i: (i, 0)),
        out_shape=jax.ShapeDtypeStruct((p_pad, 2 * D), jnp.int32),
    )(rbf, wg)

    mesh = plsc.VectorSubcoreMesh(core_axis_name="c", subcore_axis_name="s")
    acc_flat = pl.kernel(
        functools.partial(_sc_body, n_pad, p_pad, nt),
        mesh=mesh,
        out_type=jax.ShapeDtypeStruct((n_pad * ACC_W,), jnp.float32),
        scratch_types=[
            pltpu.VMEM((noffs,), jnp.int32),
            pltpu.VMEM((CH,), jnp.int32),
            pltpu.VMEM((CH,), jnp.int32),
            pltpu.VMEM((CH, 2 * D), jnp.int32),
            pltpu.VMEM((CH, 2 * D), jnp.int32),
            pltpu.VMEM((CH, 2 * D), jnp.int32),
            pltpu.VMEM((CH, 2 * D), jnp.int32),
            pltpu.VMEM((CH, 16), jnp.float32),
            pltpu.VMEM((CH, 16), jnp.float32),
            pltpu.VMEM((T_NODES * ACC_W,), jnp.float32),
            pltpu.SemaphoreType.DMA,
            pltpu.SemaphoreType.DMA,
            pltpu.SemaphoreType.DMA,
            pltpu.SemaphoreType.DMA,
        ],
    )(offs, idx_j, xcat, gcat, pdq)
    acc = acc_flat.reshape(n_pad, ACC_W)

    pptq = proj_p.T
    pdtq = proj_d.T
    fin_in = (_mlp_args(mlp_o))
    out = pl.pallas_call(
        _final_body,
        grid=(grid_n,),
        in_specs=[pl.BlockSpec((NBLK, ACC_W), lambda i: (i, 0)), row_spec,
                  full(pptq.shape), full(pdtq.shape)]
                 + [full(a.shape) for a in fin_in],
        out_specs=row_spec,
        out_shape=jax.ShapeDtypeStruct((n_pad, D), jnp.float32),
    )(acc, xx, pptq, pdtq, *fin_in)

    return out[:n]


# speedup vs baseline: 1.1098x; 1.0005x over previous
import functools

import jax
import jax.numpy as jnp
from jax import lax
from jax.experimental import pallas as pl
from jax.experimental.pallas import tpu as pltpu
from jax.experimental.pallas import tpu_sc as plsc

D = 128
R = 16
T_NODES = 32
CH = 64
NC = 2
NS = 16
NW = NC * NS
ACC_W = 9 * D
NBLK = 256
GBLK = 1000


def _swish(v):
    return v * jax.nn.sigmoid(v)


def _bf16_bits(y):
    u = lax.bitcast_convert_type(y, jnp.int32)
    return (u + 0x7FFF + ((u >> 16) & 1)) >> 16


def _pack_pairs(a, b, c):
    w1 = (_bf16_bits(a) & 0xFFFF) | (_bf16_bits(b) << 16)
    w2 = _bf16_bits(c) & 0xFFFF
    return jnp.concatenate([w1, w2], axis=1)


def _resmlp(v, w1t, b1, w2t, b2, wft, bf):
    y = _swish(v)
    y = jnp.dot(y, w1t, preferred_element_type=jnp.float32) + b1
    y = _swish(y)
    y = jnp.dot(y, w2t, preferred_element_type=jnp.float32) + b2
    v = v + y
    return jnp.dot(_swish(v), wft, preferred_element_type=jnp.float32) + bf


def _nodes_body(x_ref, *refs):
    (xw1, xb1, xw2, xb2, xwf, xbf,
     sw1, sb1, sw2, sb2, swf, sbf,
     pw1, pb1, pw2, pb2, pwf, pbf,
     dw1, db1, dw2, db2, dwf, dbf,
     xx_ref, xcat_ref) = refs
    xb = x_ref[...]
    xx = _resmlp(xb, xw1[...], xb1[...], xw2[...], xb2[...], xwf[...], xbf[...])
    xs = _resmlp(xb, sw1[...], sb1[...], sw2[...], sb2[...], swf[...], sbf[...])
    xp = _resmlp(xb, pw1[...], pb1[...], pw2[...], pb2[...], pwf[...], pbf[...])
    xd = _resmlp(xb, dw1[...], db1[...], dw2[...], db2[...], dwf[...], dbf[...])
    xx_ref[...] = xx
    xcat_ref[...] = _pack_pairs(xs, xp, xd)


def _gcat_body(rbf_ref, wg_ref, gcat_ref):
    g = jnp.dot(rbf_ref[...], wg_ref[...],
                preferred_element_type=jnp.float32)
    gcat_ref[...] = _pack_pairs(g[:, :D], g[:, D:2 * D], g[:, 2 * D:])


def _sc_body(n_pad, p_pad, nt,
             offs_hbm, idxj_hbm, xcat_hbm, gcat_hbm, pd_hbm, out_hbm,
             offs_v, ij0, ij1, xg0, xg1, gg0, gg1, pd0, pd1, acc_v,
             dsem0, dsem1, jsem0, jsem1):
    del n_pad
    np_edges = p_pad
    w = lax.axis_index("s") * NC + lax.axis_index("c")
    per_w = nt // NW
    pltpu.sync_copy(offs_hbm, offs_v)
    zv = jnp.zeros((16,), jnp.float32)
    ij = (ij0, ij1)
    xg = (xg0, xg1)
    gg = (gg0, gg1)
    pdb = (pd0, pd1)
    dsem = (dsem0, dsem1)
    jsem = (jsem0, jsem1)

    def subtile(t, carry):
        st = t * NW + w

        def zbody(z, c2):
            acc_v[pl.ds(z * 16, 16)] = zv
            return c2
        lax.fori_loop(0, (T_NODES * ACC_W) // 16, zbody, 0)

        e_lo = offs_v[pl.ds(st, 16)][0]
        e_hi = offs_v[pl.ds(st + 1, 16)][0]
        base0 = (e_lo // 8) * 8
        nch = (e_hi - base0 + CH - 1) // CH

        def _cb(k):
            return jnp.minimum(base0 + k * CH, np_edges - CH)

        def pre_j(k, b):
            pltpu.async_copy(idxj_hbm.at[pl.ds(_cb(k), CH)],
                             ij[b], jsem[b])

        def wait_j(b):
            pltpu.make_async_copy(idxj_hbm.at[pl.ds(0, CH)],
                                  ij[b], jsem[b]).wait()

        def issue_data(k, b):
            cb = _cb(k)
            pltpu.async_copy(xcat_hbm.at[ij[b]], xg[b], dsem[b])
            pltpu.async_copy(gcat_hbm.at[pl.ds(cb, CH)], gg[b], dsem[b])
            pltpu.async_copy(pd_hbm.at[pl.ds(cb, CH)], pdb[b], dsem[b])

        def wait_data(b):
            pltpu.make_async_copy(xcat_hbm.at[ij[b]], xg[b], dsem[b]).wait()
            pltpu.make_async_copy(gcat_hbm.at[pl.ds(0, CH)],
                                  gg[b], dsem[b]).wait()
            pltpu.make_async_copy(pd_hbm.at[pl.ds(0, CH)],
                                  pdb[b], dsem[b]).wait()

        def compute(k, b):
            cb_raw = base0 + k * CH
            cb = _cb(k)
            xg_v = xg[b]
            gpd_v = gg[b]
            pd_v = pdb[b]
            lo = jnp.maximum(e_lo, cb_raw) - cb
            hi = jnp.minimum(e_hi, cb_raw + CH) - cb
            cab0 = pd_v[jnp.minimum(lo, CH - 1)][3].astype(jnp.int32) * ACC_W

            for grp in ((0, 1), (2, 3), (4, 5), (6, 7)):
                offs16 = tuple(g * 16 for g in grp)
                UB = len(grp)
                NA = 9 * UB

                def edge(e, c4):
                    acc = c4[:NA]
                    cab = c4[NA]
                    pdr = pd_v[e]
                    isnew = pdr[4] > 0.5

                    @pl.when(isnew)
                    def _flush():
                        for u in range(UB):
                            for s9 in range(9):
                                plsc.addupdate(
                                    acc_v.at[pl.ds(cab + s9 * D + offs16[u],
                                                   16)],
                                    acc[u * 9 + s9])

                    ab = pdr[3].astype(jnp.int32) * ACC_W
                    cab_n = jnp.where(isnew, ab, cab)
                    sc = (pdr[0], pdr[1], pdr[2],
                          pdr[8], pdr[9], pdr[10], pdr[11], pdr[12])
                    hmask = jnp.int32(-65536)

                    def _lo(word):
                        return lax.bitcast_convert_type(word << 16,
                                                        jnp.float32)

                    def _hi(word):
                        return lax.bitcast_convert_type(word & hmask,
                                                        jnp.float32)

                    na = []
                    for u in range(UB):
                        oo = offs16[u]
                        half = u * 9
                        w_xsp = xg_v[e, pl.ds(oo, 16)]
                        w_gsp = gpd_v[e, pl.ds(oo, 16)]
                        w_xd = xg_v[e, pl.ds(D + oo, 16)]
                        w_gd = gpd_v[e, pl.ds(D + oo, 16)]
                        hs = _lo(w_xsp) * _lo(w_gsp)
                        hp = _hi(w_xsp) * _hi(w_gsp)
                        hd = _lo(w_xd) * _lo(w_gd)
                        na.append(jnp.where(isnew, zv, acc[half]) + hs)
                        for c3i in range(3):
                            na.append(jnp.where(isnew, zv, acc[half + 1 + c3i])
                                      + sc[c3i] * hp)
                        for c5i in range(5):
                            na.append(jnp.where(isnew, zv, acc[half + 4 + c5i])
                                      + sc[3 + c5i] * hd)
                    return tuple(na) + (cab_n,)

                fa = lax.fori_loop(
                    lo, hi, edge,
                    (zv,) * NA + (cab0,))
                for u in range(UB):
                    for s9 in range(9):
                        plsc.addupdate(
                            acc_v.at[pl.ds(fa[NA] + s9 * D + offs16[u], 16)],
                            fa[u * 9 + s9])

        @pl.when(nch > 0)
        def _prologue():
            pre_j(0, 0)

            @pl.when(nch > 1)
            def _pro2():
                pre_j(1, 1)

            wait_j(0)
            issue_data(0, 0)

        def pair(q, c3):
            k0 = 2 * q
            k1 = k0 + 1

            @pl.when(k1 < nch)
            def _iss1():
                wait_j(1)
                issue_data(k1, 1)

            wait_data(0)

            @pl.when(k0 + 2 < nch)
            def _pj0():
                pre_j(k0 + 2, 0)

            compute(k0, 0)

            @pl.when(k1 < nch)
            def _phase_b():
                wait_data(1)

                @pl.when(k1 + 2 < nch)
                def _pj1():
                    pre_j(k1 + 2, 1)

                @pl.when(k1 + 1 < nch)
                def _iss0():
                    wait_j(0)
                    issue_data(k1 + 1, 0)

                compute(k1, 1)
            return c3
        lax.fori_loop(0, (nch + 1) // 2, pair, 0)
        pltpu.sync_copy(
            acc_v, out_hbm.at[pl.ds(st * T_NODES * ACC_W, T_NODES * ACC_W)])
        return carry
    lax.fori_loop(0, per_w, subtile, 0)


def _final_body(acc_ref, xx_ref, ppt_ref, pdt_ref,
                ow1, ob1, ow2, ob2, owf, obf, out_ref):
    acc = acc_ref[...]
    ppt = ppt_ref[...]
    pdt = pdt_ref[...]
    s = xx_ref[...] + acc[:, 0:D]
    tot = s
    for c in range(3):
        pc = acc[:, D + c * D: D + (c + 1) * D]
        pr = jnp.dot(pc, ppt, preferred_element_type=jnp.float32)
        tot = tot + pr[:, :D] * pr[:, D:]
    for c in range(5):
        dc = acc[:, 4 * D + c * D: 4 * D + (c + 1) * D]
        dr = jnp.dot(dc, pdt, preferred_element_type=jnp.float32)
        tot = tot + dr[:, :D] * dr[:, D:]
    out_ref[...] = _resmlp(tot, ow1[...], ob1[...], ow2[...], ob2[...],
                           owf[...], obf[...])


def _mlp_args(p):
    return (p['W1'].T, p['b1'].reshape(1, D), p['W2'].T, p['b2'].reshape(1, D),
            p['Wf'].T, p['bf'].reshape(1, D))


def kernel(x, rbf, pij, dij, rs_W, rp_W, rd_W, proj_p, proj_d,
           mlp_x, mlp_s, mlp_p, mlp_d, mlp_o, idx_i, idx_j):
    n = x.shape[0]
    p = idx_i.shape[0]

    nt = -(-n // T_NODES)
    nt = -(-nt // NW) * NW
    n_pad = nt * T_NODES
    n_pad = -(-n_pad // NBLK) * NBLK
    nt = n_pad // T_NODES
    p_pad = p

    xq = jnp.pad(x, ((0, n_pad - n), (0, 0)))
    li_f = (idx_i % T_NODES).astype(jnp.float32)[:, None]
    new_f = jnp.concatenate(
        [jnp.ones((1,), jnp.float32),
         (jnp.diff(idx_i) != 0).astype(jnp.float32)])[:, None]
    pdq = jnp.concatenate(
        [pij, li_f, new_f, jnp.zeros((p, 3), jnp.float32),
         dij, jnp.zeros((p, 3), jnp.float32)], axis=1)
    del li_f, new_f
    bounds = jnp.arange(nt + 1, dtype=jnp.int32) * T_NODES
    offs = jnp.searchsorted(idx_i, bounds, side='left').astype(jnp.int32)
    noffs = -(-(nt + 1 + 16) // 16) * 16
    offs = jnp.pad(offs, (0, noffs - (nt + 1)), constant_values=p)

    wg = jnp.concatenate([rs_W, rp_W, rd_W], axis=0).T

    mlp_in = (_mlp_args(mlp_x) + _mlp_args(mlp_s) + _mlp_args(mlp_p)
              + _mlp_args(mlp_d))
    grid_n = n_pad // NBLK
    row_spec = pl.BlockSpec((NBLK, D), lambda i: (i, 0))
    full = lambda shp: pl.BlockSpec(shp, lambda i: tuple(0 for _ in shp))
    xx, xcat = pl.pallas_call(
        _nodes_body,
        grid=(grid_n,),
        in_specs=[row_spec] + [full(a.shape) for a in mlp_in],
        out_specs=[row_spec, pl.BlockSpec((NBLK, 2 * D), lambda i: (i, 0))],
        out_shape=[jax.ShapeDtypeStruct((n_pad, D), jnp.float32),
                   jax.ShapeDtypeStruct((n_pad, 2 * D), jnp.int32)],
    )(xq, *mlp_in)

    grid_p = p_pad // GBLK
    gcat = pl.pallas_call(
        _gcat_body,
        grid=(grid_p,),
        in_specs=[pl.BlockSpec((GBLK, R), lambda i: (i, 0)), full(wg.shape)],
        out_specs=pl.BlockSpec((GBLK, 2 * D), lambda i: (i, 0)),
        out_shape=jax.ShapeDtypeStruct((p_pad, 2 * D), jnp.int32),
    )(rbf, wg)

    mesh = plsc.VectorSubcoreMesh(core_axis_name="c", subcore_axis_name="s")
    acc_flat = pl.kernel(
        functools.partial(_sc_body, n_pad, p_pad, nt),
        mesh=mesh,
        out_type=jax.ShapeDtypeStruct((n_pad * ACC_W,), jnp.float32),
        scratch_types=[
            pltpu.VMEM((noffs,), jnp.int32),
            pltpu.VMEM((CH,), jnp.int32),
            pltpu.VMEM((CH,), jnp.int32),
            pltpu.VMEM((CH, 2 * D), jnp.int32),
            pltpu.VMEM((CH, 2 * D), jnp.int32),
            pltpu.VMEM((CH, 2 * D), jnp.int32),
            pltpu.VMEM((CH, 2 * D), jnp.int32),
            pltpu.VMEM((CH, 16), jnp.float32),
            pltpu.VMEM((CH, 16), jnp.float32),
            pltpu.VMEM((T_NODES * ACC_W,), jnp.float32),
            pltpu.SemaphoreType.DMA,
            pltpu.SemaphoreType.DMA,
            pltpu.SemaphoreType.DMA,
            pltpu.SemaphoreType.DMA,
        ],
    )(offs, idx_j, xcat, gcat, pdq)
    acc = acc_flat.reshape(n_pad, ACC_W)

    pptq = proj_p.T
    pdtq = proj_d.T
    fin_in = (_mlp_args(mlp_o))
    out = pl.pallas_call(
        _final_body,
        grid=(grid_n,),
        in_specs=[pl.BlockSpec((NBLK, ACC_W), lambda i: (i, 0)), row_spec,
                  full(pptq.shape), full(pdtq.shape)]
                 + [full(a.shape) for a in fin_in],
        out_specs=row_spec,
        out_shape=jax.ShapeDtypeStruct((n_pad, D), jnp.float32),
    )(acc, xx, pptq, pdtq, *fin_in)

    return out[:n]
